# baseline (speedup 1.0000x reference)
import jax
import jax.numpy as jnp
from jax.experimental import pallas as pl
from jax.experimental.pallas import tpu as pltpu

PRE_SEQ_LEN = 128
BATCH = 16
EMB_DIM = 18432
N_ROWS = BATCH * PRE_SEQ_LEN
SUB = 144
LANE = 128
WINDOW = 32
UNROLL = 4


def _gather_body(idx_ref, tbl_hbm, out_hbm, tbl_vmem, sem_t, sems):
    cp_t = pltpu.make_async_copy(tbl_hbm, tbl_vmem, sem_t)
    cp_t.start()
    cp_t.wait()

    def row_copy(i):
        idx = idx_ref[i]
        return pltpu.make_async_copy(
            tbl_vmem.at[pl.ds(idx, 1)],
            out_hbm.at[pl.ds(i, 1)],
            sems.at[i % WINDOW],
        )

    def issue(i, carry):
        for j in range(UNROLL):
            k = i * UNROLL + j

            @pl.when(k >= WINDOW)
            def _():
                row_copy(k - WINDOW).wait()

            row_copy(k).start()
        return carry

    jax.lax.fori_loop(0, N_ROWS // UNROLL, issue, 0)

    def drain(i, carry):
        row_copy(N_ROWS - WINDOW + i).wait()
        return carry

    jax.lax.fori_loop(0, WINDOW, drain, 0)


def kernel(prefix, embedding_table):
    flat_idx = prefix.reshape(N_ROWS)
    tbl = embedding_table.reshape(PRE_SEQ_LEN, SUB, LANE)

    grid_spec = pltpu.PrefetchScalarGridSpec(
        num_scalar_prefetch=1,
        grid=(1,),
        in_specs=[pl.BlockSpec(memory_space=pl.ANY)],
        out_specs=pl.BlockSpec(memory_space=pl.ANY),
        scratch_shapes=[
            pltpu.VMEM((PRE_SEQ_LEN, SUB, LANE), jnp.float32),
            pltpu.SemaphoreType.DMA,
            pltpu.SemaphoreType.DMA((WINDOW,)),
        ],
    )

    out = pl.pallas_call(
        _gather_body,
        grid_spec=grid_spec,
        out_shape=jax.ShapeDtypeStruct((N_ROWS, SUB, LANE), jnp.float32),
    )(flat_idx, tbl)
    return out.reshape(BATCH, PRE_SEQ_LEN, EMB_DIM)
